# Initial kernel scaffold; baseline (speedup 1.0000x reference)
import jax
import jax.numpy as jnp
from jax.experimental import pallas as pl


def kernel(h, edge_index, W, b):
    raise NotImplementedError("write your pallas kernel here")


# speedup vs baseline: 4.8263x; 4.8263x over previous
import functools

import jax
import jax.numpy as jnp
from jax import lax
from jax.experimental import pallas as pl
from jax.experimental.pallas import tpu as pltpu
from jax.experimental.pallas import tpu_sc as plsc

N_NODES = 10000
N_EDGES = 320000
D_IN = 128
D_OUT = 128

NC = 2
NS = 16
DH = D_IN // NC
CHUNK = 128
NPAD = 10112
ROWS_PER_TILE = NPAD // NS
NCHUNK = -(-N_EDGES // (NS * CHUNK))
HALF_CHUNK = NCHUNK // 2
EPAD = NS * NCHUNK * CHUNK
DEGPAD = 640


def _sc_segment_sum(hstk, src3, dst3):
  mesh = plsc.VectorSubcoreMesh(core_axis_name="c", subcore_axis_name="s")

  @functools.partial(
      pl.kernel,
      mesh=mesh,
      compiler_params=pltpu.CompilerParams(use_tc_tiling_on_sc=False),
      out_type=[
          jax.ShapeDtypeStruct((NC, NPAD, DH), jnp.float32),
          jax.ShapeDtypeStruct((NC, NS, ROWS_PER_TILE), jnp.float32),
      ],
      scratch_types=[
          pltpu.VMEM((CHUNK,), jnp.int32),
          pltpu.VMEM((CHUNK,), jnp.int32),
          pltpu.VMEM((CHUNK, DH), jnp.float32),
          pltpu.VMEM((CHUNK,), jnp.float32),
          pltpu.VMEM((DEGPAD,), jnp.float32),
          pltpu.VMEM_SHARED((NPAD, DH), jnp.float32),
          pltpu.VMEM_SHARED((NPAD,), jnp.float32),
          pltpu.SemaphoreType.DMA,
      ],
  )
  def k(h_hbm, src_hbm, dst_hbm, sum_out, deg_out,
        src_v, dst_v, rows_v, ones_v, degbuf, acc_sh, degacc_sh, sem):
    c = lax.axis_index("c")
    s = lax.axis_index("s")
    base = s * ROWS_PER_TILE
    row_off = c * N_NODES

    zeros16 = jnp.zeros((16,), jnp.float32)

    def zrow(i, _):
      for kk in range(DH // 16):
        rows_v[i, pl.ds(kk * 16, 16)] = zeros16
      return 0
    lax.fori_loop(0, CHUNK, zrow, 0)

    def zone(i, _):
      ones_v[pl.ds(i * 16, 16)] = jnp.ones((16,), jnp.float32)
      return 0
    lax.fori_loop(0, CHUNK // 16, zone, 0)

    def zdeg(i, _):
      degbuf[pl.ds(i * 16, 16)] = zeros16
      return 0
    lax.fori_loop(0, DEGPAD // 16, zdeg, 0)

    nfull = ROWS_PER_TILE // CHUNK
    rem = ROWS_PER_TILE - nfull * CHUNK
    for kk in range(nfull):
      pltpu.sync_copy(rows_v, acc_sh.at[pl.ds(base + kk * CHUNK, CHUNK)])
    pltpu.sync_copy(rows_v.at[pl.ds(0, rem)],
                    acc_sh.at[pl.ds(base + nfull * CHUNK, rem)])
    pltpu.sync_copy(degbuf.at[pl.ds(0, ROWS_PER_TILE)],
                    degacc_sh.at[pl.ds(base, ROWS_PER_TILE)])

    plsc.subcore_barrier()

    def body(j, _):
      pltpu.sync_copy(src_hbm.at[s, j], src_v)
      pltpu.sync_copy(dst_hbm.at[s, j], dst_v)
      for kk in range(CHUNK // 16):
        sl = pl.ds(kk * 16, 16)
        src_v[sl] = src_v[sl] + jnp.broadcast_to(row_off, (16,))
      pltpu.async_copy(h_hbm.at[src_v], rows_v, sem).wait()
      pltpu.sync_copy(rows_v, acc_sh.at[dst_v], add=True)

      do_deg = jnp.where(c == 0, j < HALF_CHUNK, j >= HALF_CHUNK)
      @pl.when(do_deg)
      def _():
        pltpu.sync_copy(ones_v, degacc_sh.at[dst_v], add=True)
      return 0
    lax.fori_loop(0, NCHUNK, body, 0)

    plsc.subcore_barrier()

    for kk in range(nfull):
      pltpu.sync_copy(acc_sh.at[pl.ds(base + kk * CHUNK, CHUNK)], rows_v)
      pltpu.sync_copy(rows_v, sum_out.at[c, pl.ds(base + kk * CHUNK, CHUNK)])
    pltpu.sync_copy(acc_sh.at[pl.ds(base + nfull * CHUNK, rem)],
                    rows_v.at[pl.ds(0, rem)])
    pltpu.sync_copy(rows_v.at[pl.ds(0, rem)],
                    sum_out.at[c, pl.ds(base + nfull * CHUNK, rem)])
    pltpu.sync_copy(degacc_sh.at[pl.ds(base, ROWS_PER_TILE)],
                    degbuf.at[pl.ds(0, ROWS_PER_TILE)])
    pltpu.sync_copy(degbuf.at[pl.ds(0, ROWS_PER_TILE)], deg_out.at[c, s])

  return k(hstk, src3, dst3)


def _tc_combine(h, sums, degs, wt, b2):
  R = 2000
  grid = (N_NODES // R,)

  def body(h_ref, p_ref, d_ref, wt_ref, b_ref, o_ref):
    deg = d_ref[0] + d_ref[1]
    inv = 1.0 / jnp.maximum(deg, 1.0)
    hn = jnp.concatenate([p_ref[0], p_ref[1]], axis=1) * inv
    acc = jnp.dot(h_ref[...], wt_ref[0:D_IN, :],
                  preferred_element_type=jnp.float32)
    acc += jnp.dot(hn, wt_ref[D_IN:2 * D_IN, :],
                   preferred_element_type=jnp.float32)
    o_ref[...] = acc + b_ref[...]

  return pl.pallas_call(
      body,
      grid=grid,
      in_specs=[
          pl.BlockSpec((R, D_IN), lambda i: (i, 0)),
          pl.BlockSpec((NC, R, DH), lambda i: (0, i, 0)),
          pl.BlockSpec((NC, R, 1), lambda i: (0, i, 0)),
          pl.BlockSpec((2 * D_IN, D_OUT), lambda i: (0, 0)),
          pl.BlockSpec((1, D_OUT), lambda i: (0, 0)),
      ],
      out_specs=pl.BlockSpec((R, D_OUT), lambda i: (i, 0)),
      out_shape=jax.ShapeDtypeStruct((N_NODES, D_OUT), jnp.float32),
  )(h, sums, degs, wt, b2)


def kernel(h, edge_index, W, b):
  src = edge_index[0].astype(jnp.int32)
  dst = edge_index[1].astype(jnp.int32)
  npad_e = EPAD - N_EDGES
  src = jnp.concatenate([src, jnp.zeros((npad_e,), jnp.int32)])
  dst = jnp.concatenate([dst, jnp.full((npad_e,), N_NODES, jnp.int32)])
  src3 = src.reshape(NS, NCHUNK, CHUNK)
  dst3 = dst.reshape(NS, NCHUNK, CHUNK)
  hstk = jnp.concatenate([h[:, :DH], h[:, DH:]], axis=0)

  sums, degs = _sc_segment_sum(hstk, src3, dst3)
  degs = degs.reshape(NC, NPAD, 1)

  wt = W.T
  b2 = b.reshape(1, D_OUT)
  return _tc_combine(h, sums, degs, wt, b2)


# speedup vs baseline: 6.0944x; 1.2627x over previous
import functools

import jax
import jax.numpy as jnp
from jax import lax
from jax.experimental import pallas as pl
from jax.experimental.pallas import tpu as pltpu
from jax.experimental.pallas import tpu_sc as plsc

N_NODES = 10000
N_EDGES = 320000
D_IN = 128
D_OUT = 128

NC = 2
NS = 16
DH = D_IN // NC
CHUNK = 128
NPAD = 10112
ROWS_PER_TILE = NPAD // NS
NB = 4
NCHUNK = 160
NR = NCHUNK // NB
EPAD = NS * NCHUNK * CHUNK
DEGPAD = 640


def _sc_segment_sum(hstk, src3, dst3):
  mesh = plsc.VectorSubcoreMesh(core_axis_name="c", subcore_axis_name="s")

  @functools.partial(
      pl.kernel,
      mesh=mesh,
      compiler_params=pltpu.CompilerParams(use_tc_tiling_on_sc=False),
      out_type=[
          jax.ShapeDtypeStruct((NC, NPAD, DH), jnp.float32),
          jax.ShapeDtypeStruct((NC, NS, ROWS_PER_TILE), jnp.float32),
      ],
      scratch_types=(
          [
              pltpu.VMEM((2, NCHUNK, CHUNK), jnp.int32),
              pltpu.VMEM((NB * CHUNK, DH), jnp.float32),
              pltpu.VMEM((DEGPAD + CHUNK,), jnp.float32),
              pltpu.VMEM_SHARED((NPAD, DH), jnp.float32),
              pltpu.VMEM_SHARED((NPAD,), jnp.float32),
          ]
          + [pltpu.SemaphoreType.DMA] * (3 * NB)
      ),
  )
  def k(h_hbm, src_hbm, dst_hbm, sum_out, deg_out,
        idxbuf, rowsbuf, fbuf, acc_sh, degacc_sh, *sems):
    rows = [rowsbuf.at[pl.ds(b * CHUNK, CHUNK)] for b in range(NB)]
    ones_v = fbuf.at[pl.ds(DEGPAD, CHUNK)]
    degbuf = fbuf.at[pl.ds(0, DEGPAD)]
    gsem = sems[:NB]
    ssem = sems[NB:2 * NB]
    dsem = sems[2 * NB:3 * NB]
    c = lax.axis_index("c")
    s = lax.axis_index("s")
    base = s * ROWS_PER_TILE

    zeros16 = jnp.zeros((16,), jnp.float32)

    pltpu.sync_copy(src_hbm.at[c, s], idxbuf.at[0])
    pltpu.sync_copy(dst_hbm.at[s], idxbuf.at[1])

    def zrow(i, _):
      for kk in range(DH // 16):
        rowsbuf[i, pl.ds(kk * 16, 16)] = zeros16
      return 0
    lax.fori_loop(0, CHUNK, zrow, 0)

    def zfb(i, _):
      fbuf[pl.ds(i * 16, 16)] = zeros16
      return 0
    lax.fori_loop(0, DEGPAD // 16, zfb, 0)

    def zone(i, _):
      fbuf[pl.ds(DEGPAD + i * 16, 16)] = jnp.ones((16,), jnp.float32)
      return 0
    lax.fori_loop(0, CHUNK // 16, zone, 0)

    nfull = ROWS_PER_TILE // CHUNK
    rem = ROWS_PER_TILE - nfull * CHUNK
    for kk in range(nfull):
      pltpu.sync_copy(rowsbuf.at[pl.ds(0, CHUNK)],
                      acc_sh.at[pl.ds(base + kk * CHUNK, CHUNK)])
    pltpu.sync_copy(rowsbuf.at[pl.ds(0, rem)],
                    acc_sh.at[pl.ds(base + nfull * CHUNK, rem)])
    pltpu.sync_copy(degbuf.at[pl.ds(0, ROWS_PER_TILE)],
                    degacc_sh.at[pl.ds(base, ROWS_PER_TILE)])

    plsc.subcore_barrier()

    for b in range(NB):
      pltpu.async_copy(h_hbm.at[idxbuf.at[0, b]], rows[b], gsem[b])

    def rnd(g, _):
      j0 = g * NB
      for b in range(NB):
        pltpu.make_async_copy(h_hbm.at[pl.ds(0, CHUNK)], rows[b],
                              gsem[b]).wait()
        pltpu.async_copy(rows[b], acc_sh.at[idxbuf.at[1, j0 + b]], ssem[b],
                         add=True)
        pltpu.async_copy(ones_v, degacc_sh.at[idxbuf.at[1, j0 + b]], dsem[b],
                         add=True)
      for b in range(NB):
        pltpu.make_async_copy(rows[b], acc_sh.at[pl.ds(0, CHUNK)],
                              ssem[b]).wait()
        pltpu.make_async_copy(ones_v, degacc_sh.at[pl.ds(0, CHUNK)],
                              dsem[b]).wait()
        @pl.when(g < NR - 1)
        def _():
          pltpu.async_copy(h_hbm.at[idxbuf.at[0, j0 + NB + b]], rows[b],
                           gsem[b])
      return 0
    lax.fori_loop(0, NR, rnd, 0)

    plsc.subcore_barrier()

    for kk in range(nfull):
      pltpu.sync_copy(acc_sh.at[pl.ds(base + kk * CHUNK, CHUNK)],
                      rowsbuf.at[pl.ds(0, CHUNK)])
      pltpu.sync_copy(rowsbuf.at[pl.ds(0, CHUNK)],
                      sum_out.at[c, pl.ds(base + kk * CHUNK, CHUNK)])
    pltpu.sync_copy(acc_sh.at[pl.ds(base + nfull * CHUNK, rem)],
                    rowsbuf.at[pl.ds(0, rem)])
    pltpu.sync_copy(rowsbuf.at[pl.ds(0, rem)],
                    sum_out.at[c, pl.ds(base + nfull * CHUNK, rem)])
    pltpu.sync_copy(degacc_sh.at[pl.ds(base, ROWS_PER_TILE)],
                    degbuf.at[pl.ds(0, ROWS_PER_TILE)])
    pltpu.sync_copy(degbuf.at[pl.ds(0, ROWS_PER_TILE)], deg_out.at[c, s])

  return k(hstk, src3, dst3)


def _tc_combine(h, sums, degs, wt, b2):
  R = 2000
  grid = (N_NODES // R,)

  def body(h_ref, p_ref, d_ref, wt_ref, b_ref, o_ref):
    deg = (d_ref[0] + d_ref[1]) * 0.5
    inv = 1.0 / jnp.maximum(deg, 1.0)
    hn = jnp.concatenate([p_ref[0], p_ref[1]], axis=1) * inv
    acc = jnp.dot(h_ref[...], wt_ref[0:D_IN, :],
                  preferred_element_type=jnp.float32)
    acc += jnp.dot(hn, wt_ref[D_IN:2 * D_IN, :],
                   preferred_element_type=jnp.float32)
    o_ref[...] = acc + b_ref[...]

  return pl.pallas_call(
      body,
      grid=grid,
      in_specs=[
          pl.BlockSpec((R, D_IN), lambda i: (i, 0)),
          pl.BlockSpec((NC, R, DH), lambda i: (0, i, 0)),
          pl.BlockSpec((NC, R, 1), lambda i: (0, i, 0)),
          pl.BlockSpec((2 * D_IN, D_OUT), lambda i: (0, 0)),
          pl.BlockSpec((1, D_OUT), lambda i: (0, 0)),
      ],
      out_specs=pl.BlockSpec((R, D_OUT), lambda i: (i, 0)),
      out_shape=jax.ShapeDtypeStruct((N_NODES, D_OUT), jnp.float32),
  )(h, sums, degs, wt, b2)


def kernel(h, edge_index, W, b):
  src = edge_index[0].astype(jnp.int32)
  dst = edge_index[1].astype(jnp.int32)
  npad_e = EPAD - N_EDGES
  src = jnp.concatenate([src, jnp.zeros((npad_e,), jnp.int32)])
  dst = jnp.concatenate([dst, jnp.full((npad_e,), N_NODES, jnp.int32)])
  src3 = jnp.stack([src, src + N_NODES]).reshape(NC, NS, NCHUNK, CHUNK)
  dst3 = dst.reshape(NS, NCHUNK, CHUNK)
  hstk = jnp.concatenate([h[:, :DH], h[:, DH:]], axis=0)

  sums, degs = _sc_segment_sum(hstk, src3, dst3)
  degs = degs.reshape(NC, NPAD, 1)

  wt = W.T
  b2 = b.reshape(1, D_OUT)
  return _tc_combine(h, sums, degs, wt, b2)
